# baseline (speedup 1.0000x reference)
import functools

import jax
import jax.numpy as jnp
from jax import lax
from jax.experimental import pallas as pl
from jax.experimental.pallas import tpu as pltpu
from jax.experimental.pallas import tpu_sc as plsc

SEQ = 256
C = 1024
CH = 16
JC = SEQ // CH
NB = 3
LEAD = 1

_info = plsc.get_sparse_core_info()
_NC, _NS = _info.num_cores, _info.num_subcores
_NW = _NC * _NS
_ROWS = SEQ // _NW

_mesh = plsc.VectorSubcoreMesh(core_axis_name="c", subcore_axis_name="s")


@functools.partial(
    pl.kernel,
    mesh=_mesh,
    out_type=jax.ShapeDtypeStruct((SEQ, SEQ, 2 * C), jnp.float32),
    scratch_types=[
        pltpu.VMEM((NB, CH, 2 * C), jnp.float32),
        pltpu.SemaphoreType.DMA,
        pltpu.SemaphoreType.DMA,
        pltpu.SemaphoreType.DMA,
    ],
)
def _encode(x_hbm, emb_hbm, out_hbm, buf, xin_sem, ein_sem, out_sem):
    wid = lax.axis_index("s") * _NC + lax.axis_index("c")
    base = wid * _ROWS

    T = JC * _ROWS
    xin = [None] * NB
    ein = [None] * NB
    outh = [None] * NB
    e_content = [None] * NB

    def issue_in(t):
        p = t % NB
        jc, ii = divmod(t, _ROWS)
        xin[p] = pltpu.async_copy(
            x_hbm.at[base + ii, pl.ds(jc * CH, CH), :],
            buf.at[p, :, pl.ds(0, C)], xin_sem)
        if e_content[p] != jc:
            ein[p] = pltpu.async_copy(
                emb_hbm.at[pl.ds(jc * CH, CH), :],
                buf.at[p, :, pl.ds(C, C)], ein_sem)
            e_content[p] = jc
        else:
            ein[p] = None

    for t in range(LEAD + 1):
        issue_in(t)

    for t in range(T):
        p = t % NB
        jc, ii = divmod(t, _ROWS)
        nt = t + LEAD + 1
        if nt < T:
            r = nt % NB
            if outh[r] is not None:
                outh[r].wait()
            issue_in(nt)
        xin[p].wait()
        if ein[p] is not None:
            ein[p].wait()
        outh[p] = pltpu.async_copy(
            buf.at[p], out_hbm.at[base + ii, pl.ds(jc * CH, CH), :], out_sem)

    for h in outh:
        if h is not None:
            h.wait()


def kernel(x, embedding):
    return _encode(x, embedding)


# speedup vs baseline: 1.2342x; 1.2342x over previous
import functools

import jax
import jax.numpy as jnp
from jax import lax
from jax.experimental import pallas as pl
from jax.experimental.pallas import tpu as pltpu
from jax.experimental.pallas import tpu_sc as plsc

SEQ = 256
C = 1024
CH = 16
JC = SEQ // CH

_info = plsc.get_sparse_core_info()
_NC, _NS = _info.num_cores, _info.num_subcores
_NW = _NC * _NS
_ROWS = SEQ // _NW

_mesh = plsc.VectorSubcoreMesh(core_axis_name="c", subcore_axis_name="s")


@functools.partial(
    pl.kernel,
    mesh=_mesh,
    out_type=jax.ShapeDtypeStruct((SEQ, SEQ, 2 * C), jnp.float32),
    scratch_types=[
        pltpu.VMEM((2, CH, C), jnp.float32),
        pltpu.SemaphoreType.DMA,
        pltpu.SemaphoreType.DMA,
    ],
)
def _sc_embed(emb_hbm, out_hbm, ebuf, ein_sem, eout_sem):
    wid = lax.axis_index("s") * _NC + lax.axis_index("c")
    base = wid * _ROWS

    ein = [None, None]
    eouts = [[], []]
    ein[0] = pltpu.async_copy(emb_hbm.at[pl.ds(0, CH), :], ebuf.at[0], ein_sem)
    for jc in range(JC):
        ep = jc & 1
        ein[ep].wait()
        if jc + 1 < JC:
            for h in eouts[1 - ep]:
                h.wait()
            eouts[1 - ep] = []
            ein[1 - ep] = pltpu.async_copy(
                emb_hbm.at[pl.ds((jc + 1) * CH, CH), :], ebuf.at[1 - ep],
                ein_sem)
        for ii in range(_ROWS):
            eouts[ep].append(pltpu.async_copy(
                ebuf.at[ep],
                out_hbm.at[base + ii, pl.ds(jc * CH, CH), pl.ds(C, C)],
                eout_sem))
    for hs in eouts:
        for h in hs:
            h.wait()


def _tc_body(x_ref, shell_ref, out_ref):
    del shell_ref
    out_ref[...] = x_ref[...]


_BI = 4

_tc_fill = pl.pallas_call(
    _tc_body,
    grid=(SEQ // _BI,),
    in_specs=[
        pl.BlockSpec((_BI, SEQ, C), lambda i: (i, 0, 0)),
        pl.BlockSpec(memory_space=pl.ANY),
    ],
    out_specs=pl.BlockSpec((_BI, SEQ, C), lambda i: (i, 0, 0)),
    out_shape=jax.ShapeDtypeStruct((SEQ, SEQ, 2 * C), jnp.float32),
    input_output_aliases={1: 0},
)


def kernel(x, embedding):
    shell = _sc_embed(embedding)
    return _tc_fill(x, shell)


# speedup vs baseline: 1.2494x; 1.0123x over previous
import functools

import jax
import jax.numpy as jnp
from jax import lax
from jax.experimental import pallas as pl
from jax.experimental.pallas import tpu as pltpu
from jax.experimental.pallas import tpu_sc as plsc

SEQ = 256
C = 1024
CH = 32
JC = SEQ // CH

_info = plsc.get_sparse_core_info()
_NC, _NS = _info.num_cores, _info.num_subcores
_NW = _NC * _NS
_ROWS = SEQ // _NW

_mesh = plsc.VectorSubcoreMesh(core_axis_name="c", subcore_axis_name="s")


@functools.partial(
    pl.kernel,
    mesh=_mesh,
    out_type=jax.ShapeDtypeStruct((SEQ, SEQ, 2 * C), jnp.float32),
    scratch_types=[
        pltpu.VMEM((2, CH, C), jnp.float32),
        pltpu.SemaphoreType.DMA,
        pltpu.SemaphoreType.DMA,
    ],
)
def _sc_embed(emb_hbm, out_hbm, ebuf, ein_sem, eout_sem):
    wid = lax.axis_index("s") * _NC + lax.axis_index("c")
    base = wid * _ROWS

    ein = [None, None]
    eouts = [[], []]
    ein[0] = pltpu.async_copy(emb_hbm.at[pl.ds(0, CH), :], ebuf.at[0], ein_sem)
    for jc in range(JC):
        ep = jc & 1
        ein[ep].wait()
        if jc + 1 < JC:
            for h in eouts[1 - ep]:
                h.wait()
            eouts[1 - ep] = []
            ein[1 - ep] = pltpu.async_copy(
                emb_hbm.at[pl.ds((jc + 1) * CH, CH), :], ebuf.at[1 - ep],
                ein_sem)
        for ii in range(_ROWS):
            eouts[ep].append(pltpu.async_copy(
                ebuf.at[ep],
                out_hbm.at[base + ii, pl.ds(jc * CH, CH), pl.ds(C, C)],
                eout_sem))
    for hs in eouts:
        for h in hs:
            h.wait()


def _tc_body(x_ref, shell_ref, out_ref):
    del shell_ref
    out_ref[...] = x_ref[...]


_BI = 8

_tc_fill = pl.pallas_call(
    _tc_body,
    grid=(SEQ // _BI,),
    in_specs=[
        pl.BlockSpec((_BI, SEQ, C), lambda i: (i, 0, 0)),
        pl.BlockSpec(memory_space=pl.ANY),
    ],
    out_specs=pl.BlockSpec((_BI, SEQ, C), lambda i: (i, 0, 0)),
    out_shape=jax.ShapeDtypeStruct((SEQ, SEQ, 2 * C), jnp.float32),
    input_output_aliases={1: 0},
)


def kernel(x, embedding):
    shell = _sc_embed(embedding)
    return _tc_fill(x, shell)
